# Initial kernel scaffold; baseline (speedup 1.0000x reference)
import jax
import jax.numpy as jnp
from jax.experimental import pallas as pl


def kernel(x, table):
    raise NotImplementedError("write your pallas kernel here")


# speedup vs baseline: 4.7635x; 4.7635x over previous
import functools
import math

import jax
import jax.numpy as jnp
from jax import lax
from jax.experimental import pallas as pl
from jax.experimental.pallas import tpu as pltpu
from jax.experimental.pallas import tpu_sc as plsc


def _scale_block(t_ref, o_ref):
    o_ref[...] = t_ref[...] * o_ref.shape[-1] ** 0.5


def _scale_table(table):
    v, d = table.shape
    block = 2000
    return pl.pallas_call(
        _scale_block,
        out_shape=jax.ShapeDtypeStruct((v, d), table.dtype),
        grid=(v // block,),
        in_specs=[pl.BlockSpec((block, d), lambda i: (i, 0))],
        out_specs=pl.BlockSpec((block, d), lambda i: (i, 0)),
    )(table)


@functools.cache
def _make_gather(B, D):
    num_cores, num_subcores = 2, 16
    nw = num_cores * num_subcores
    bpw = B // nw
    chunk = 128
    nchunk = bpw // chunk
    mesh = plsc.VectorSubcoreMesh(core_axis_name="c", subcore_axis_name="s")

    @functools.partial(
        pl.kernel,
        mesh=mesh,
        out_type=jax.ShapeDtypeStruct((B, D), jnp.float32),
        scratch_types=[
            pltpu.VMEM((chunk,), jnp.int32),
            pltpu.VMEM((chunk, D), jnp.float32),
            pltpu.SemaphoreType.DMA,
        ],
    )
    def gather(table_hbm, idx_hbm, out_hbm, idx_v, rows_v, sem):
        wid = lax.axis_index("s") * num_cores + lax.axis_index("c")
        base = wid * bpw

        def body(g, carry):
            off = base + g * chunk
            pltpu.sync_copy(idx_hbm.at[pl.ds(off, chunk)], idx_v)
            pltpu.async_copy(table_hbm.at[idx_v], rows_v, sem).wait()
            pltpu.sync_copy(rows_v, out_hbm.at[pl.ds(off, chunk)])
            return carry

        lax.fori_loop(0, nchunk, body, 0)

    return gather


def kernel(x, table):
    n, s = x.shape
    d = table.shape[1]
    b = n * s
    scaled = _scale_table(table)
    flat = x.reshape(b).astype(jnp.int32)
    out = _make_gather(b, d)(scaled, flat)
    return out.reshape(n, s, d)


# speedup vs baseline: 7.9590x; 1.6708x over previous
import functools

import jax
import jax.numpy as jnp
from jax import lax
from jax.experimental import pallas as pl
from jax.experimental.pallas import tpu as pltpu
from jax.experimental.pallas import tpu_sc as plsc


def _scale_block(t_ref, o_ref):
    o_ref[...] = t_ref[...] * o_ref.shape[-1] ** 0.5


def _scale_table(table):
    v, d = table.shape
    block = 2000
    return pl.pallas_call(
        _scale_block,
        out_shape=jax.ShapeDtypeStruct((v, d), table.dtype),
        grid=(v // block,),
        in_specs=[pl.BlockSpec((block, d), lambda i: (i, 0))],
        out_specs=pl.BlockSpec((block, d), lambda i: (i, 0)),
    )(table)


NBUF = 4


@functools.cache
def _make_gather(B, D):
    num_cores, num_subcores = 2, 16
    nw = num_cores * num_subcores
    bpw = B // nw
    chunk = 128
    nchunk = bpw // chunk
    mesh = plsc.VectorSubcoreMesh(core_axis_name="c", subcore_axis_name="s")

    scratch = [pltpu.VMEM((nchunk, chunk), jnp.int32)]
    scratch += [pltpu.VMEM((chunk, D), jnp.float32) for _ in range(NBUF)]
    scratch += [pltpu.SemaphoreType.DMA for _ in range(2 * NBUF)]

    @functools.partial(
        pl.kernel,
        mesh=mesh,
        out_type=jax.ShapeDtypeStruct((B, D), jnp.float32),
        scratch_types=scratch,
    )
    def gather(table_hbm, idx_hbm, out_hbm, idx_all, *bufs):
        rows = bufs[:NBUF]
        gsem = bufs[NBUF : 2 * NBUF]
        ssem = bufs[2 * NBUF :]
        wid = lax.axis_index("s") * num_cores + lax.axis_index("c")
        cbase = wid * nchunk

        pltpu.sync_copy(idx_hbm.at[pl.ds(cbase, nchunk)], idx_all)

        def start_gather(l, b):
            pltpu.async_copy(table_hbm.at[idx_all.at[l]], rows[b], gsem[b])

        def wait_gather(b):
            pltpu.make_async_copy(table_hbm.at[idx_all.at[0]], rows[b], gsem[b]).wait()

        def start_scatter(l, b):
            off = (cbase + l) * chunk
            pltpu.async_copy(rows[b], out_hbm.at[pl.ds(off, chunk)], ssem[b])

        def wait_scatter(l, b):
            off = (cbase + l) * chunk
            pltpu.make_async_copy(
                rows[b], out_hbm.at[pl.ds(off, chunk)], ssem[b]
            ).wait()

        for b in range(NBUF):
            start_gather(b, b)

        def body(g, carry):
            for b in range(NBUF):
                l = g * NBUF + b
                wait_gather(b)
                start_scatter(l, b)
                wait_scatter(l, b)
                start_gather(l + NBUF, b)
            return carry

        lax.fori_loop(0, nchunk // NBUF - 1, body, 0)

        for b in range(NBUF):
            l = nchunk - NBUF + b
            wait_gather(b)
            start_scatter(l, b)
            wait_scatter(l, b)

    return gather


def kernel(x, table):
    n, s = x.shape
    d = table.shape[1]
    b = n * s
    scaled = _scale_table(table)
    idx2 = x.reshape(b // 128, 128).astype(jnp.int32)
    out = _make_gather(b, d)(scaled, idx2)
    return out.reshape(n, s, d)


# speedup vs baseline: 7.9818x; 1.0029x over previous
import functools

import jax
import jax.numpy as jnp
from jax import lax
from jax.experimental import pallas as pl
from jax.experimental.pallas import tpu as pltpu
from jax.experimental.pallas import tpu_sc as plsc


def _scale_block(t_ref, o_ref):
    o_ref[...] = t_ref[...] * o_ref.shape[-1] ** 0.5


def _scale_table(table):
    v, d = table.shape
    block = 2000
    return pl.pallas_call(
        _scale_block,
        out_shape=jax.ShapeDtypeStruct((v, d), table.dtype),
        grid=(v // block,),
        in_specs=[pl.BlockSpec((block, d), lambda i: (i, 0))],
        out_specs=pl.BlockSpec((block, d), lambda i: (i, 0)),
    )(table)


@functools.cache
def _make_gather(B, D):
    num_cores, num_subcores = 2, 16
    nw = num_cores * num_subcores
    bpw = B // nw
    chunk = 128
    nchunk = bpw // chunk
    half = 2 * chunk
    nhalf = nchunk // 2
    mesh = plsc.VectorSubcoreMesh(core_axis_name="c", subcore_axis_name="s")

    scratch = [pltpu.VMEM((nchunk, chunk), jnp.int32)]
    scratch += [pltpu.VMEM((half, D), jnp.float32) for _ in range(2)]
    scratch += [pltpu.SemaphoreType.DMA for _ in range(4)]

    @functools.partial(
        pl.kernel,
        mesh=mesh,
        out_type=jax.ShapeDtypeStruct((B, D), jnp.float32),
        scratch_types=scratch,
    )
    def gather(table_hbm, idx_hbm, out_hbm, idx_all, r0, r1, g0, g1, s0, s1):
        rows = (r0, r1)
        gsem = (g0, g1)
        ssem = (s0, s1)
        wid = lax.axis_index("s") * num_cores + lax.axis_index("c")
        cbase = wid * nchunk

        pltpu.sync_copy(idx_hbm.at[pl.ds(cbase, nchunk)], idx_all)

        def start_gathers(h, p):
            pltpu.async_copy(
                table_hbm.at[idx_all.at[2 * h]], rows[p].at[pl.ds(0, chunk)], gsem[p]
            )
            pltpu.async_copy(
                table_hbm.at[idx_all.at[2 * h + 1]],
                rows[p].at[pl.ds(chunk, chunk)],
                gsem[p],
            )

        def wait_gathers(p):
            pltpu.make_async_copy(
                table_hbm.at[idx_all.at[0]], rows[p], gsem[p]
            ).wait()

        def start_scatter(h, p):
            off = cbase * chunk + h * half
            pltpu.async_copy(rows[p], out_hbm.at[pl.ds(off, half)], ssem[p])

        def wait_scatter(h, p):
            off = cbase * chunk + h * half
            pltpu.make_async_copy(
                rows[p], out_hbm.at[pl.ds(off, half)], ssem[p]
            ).wait()

        start_gathers(0, 0)
        start_gathers(1, 1)
        wait_gathers(0)
        start_scatter(0, 0)

        def body(g, carry):
            h1 = 2 * g + 1
            wait_gathers(1)
            start_scatter(h1, 1)
            wait_scatter(h1 - 1, 0)
            start_gathers(h1 + 1, 0)
            wait_gathers(0)
            start_scatter(h1 + 1, 0)
            wait_scatter(h1, 1)
            start_gathers(h1 + 2, 1)
            return carry

        lax.fori_loop(0, nhalf // 2 - 1, body, 0)

        wait_gathers(1)
        start_scatter(nhalf - 1, 1)
        wait_scatter(nhalf - 2, 0)
        wait_scatter(nhalf - 1, 1)

    return gather


def kernel(x, table):
    n, s = x.shape
    d = table.shape[1]
    b = n * s
    scaled = _scale_table(table)
    idx2 = x.reshape(b // 128, 128).astype(jnp.int32)
    out = _make_gather(b, d)(scaled, idx2)
    return out.reshape(n, s, d)


# speedup vs baseline: 8.3728x; 1.0490x over previous
import functools

import jax
import jax.numpy as jnp
from jax import lax
from jax.experimental import pallas as pl
from jax.experimental.pallas import tpu as pltpu
from jax.experimental.pallas import tpu_sc as plsc


def _scale_block(t_ref, o_ref):
    o_ref[...] = t_ref[...] * o_ref.shape[-1] ** 0.5


def _scale_table(table):
    v, d = table.shape
    block = 10000
    return pl.pallas_call(
        _scale_block,
        out_shape=jax.ShapeDtypeStruct((v, d), table.dtype),
        grid=(v // block,),
        in_specs=[pl.BlockSpec((block, d), lambda i: (i, 0))],
        out_specs=pl.BlockSpec((block, d), lambda i: (i, 0)),
    )(table)


@functools.cache
def _make_gather(B, D):
    num_cores, num_subcores = 2, 16
    nw = num_cores * num_subcores
    bpw = B // nw
    chunk = 128
    nchunk = bpw // chunk
    half = 2 * chunk
    nhalf = nchunk // 2
    mesh = plsc.VectorSubcoreMesh(core_axis_name="c", subcore_axis_name="s")

    scratch = [pltpu.VMEM((nchunk, chunk), jnp.int32)]
    scratch += [pltpu.VMEM((half, D), jnp.float32) for _ in range(2)]
    scratch += [pltpu.SemaphoreType.DMA for _ in range(4)]

    @functools.partial(
        pl.kernel,
        mesh=mesh,
        out_type=jax.ShapeDtypeStruct((B, D), jnp.float32),
        scratch_types=scratch,
    )
    def gather(table_hbm, idx_hbm, out_hbm, idx_all, r0, r1, g0, g1, s0, s1):
        rows = (r0, r1)
        gsem = (g0, g1)
        ssem = (s0, s1)
        wid = lax.axis_index("s") * num_cores + lax.axis_index("c")
        cbase = wid * nchunk

        pltpu.sync_copy(idx_hbm.at[pl.ds(cbase, nchunk)], idx_all)

        def start_gathers(h, p):
            pltpu.async_copy(
                table_hbm.at[idx_all.at[2 * h]], rows[p].at[pl.ds(0, chunk)], gsem[p]
            )
            pltpu.async_copy(
                table_hbm.at[idx_all.at[2 * h + 1]],
                rows[p].at[pl.ds(chunk, chunk)],
                gsem[p],
            )

        def wait_gathers(p):
            pltpu.make_async_copy(
                table_hbm.at[idx_all.at[0]], rows[p], gsem[p]
            ).wait()

        def start_scatter(h, p):
            off = cbase * chunk + h * half
            pltpu.async_copy(rows[p], out_hbm.at[pl.ds(off, half)], ssem[p])

        def wait_scatter(h, p):
            off = cbase * chunk + h * half
            pltpu.make_async_copy(
                rows[p], out_hbm.at[pl.ds(off, half)], ssem[p]
            ).wait()

        start_gathers(0, 0)
        start_gathers(1, 1)
        wait_gathers(0)
        start_scatter(0, 0)

        def body(g, carry):
            h1 = 2 * g + 1
            wait_gathers(1)
            start_scatter(h1, 1)
            wait_scatter(h1 - 1, 0)
            start_gathers(h1 + 1, 0)
            wait_gathers(0)
            start_scatter(h1 + 1, 0)
            wait_scatter(h1, 1)
            start_gathers(h1 + 2, 1)
            return carry

        lax.fori_loop(0, nhalf // 2 - 1, body, 0)

        wait_gathers(1)
        start_scatter(nhalf - 1, 1)
        wait_scatter(nhalf - 2, 0)
        wait_scatter(nhalf - 1, 1)

    return gather


def kernel(x, table):
    n, s = x.shape
    d = table.shape[1]
    b = n * s
    scaled = _scale_table(table)
    idx2 = x.reshape(b // 128, 128).astype(jnp.int32)
    out = _make_gather(b, d)(scaled, idx2)
    return out.reshape(n, s, d)
